# Initial kernel scaffold; baseline (speedup 1.0000x reference)
import functools

import jax
import jax.numpy as jnp
from jax import lax
from jax.experimental import pallas as pl
from jax.experimental.pallas import tpu as pltpu
from jax.experimental.pallas import tpu_sc as plsc

VEC = 64
B = 16384
K = 26
NW = 32
BPW = B // NW
SUB = 64
NSUB = BPW // SUB
PAIRS = SUB * K
IDX_ROWS = PAIRS // 128
ROWS_PER_W = (B * K) // (NW * 128)

_mesh = plsc.VectorSubcoreMesh(core_axis_name="c", subcore_axis_name="s")


@functools.partial(
    pl.kernel,
    mesh=_mesh,
    out_type=jax.ShapeDtypeStruct((B * K,), jnp.float32),
    scratch_types=[
        pltpu.VMEM((SUB,), jnp.int32),
        pltpu.VMEM((IDX_ROWS, 128), jnp.int32),
        pltpu.VMEM((SUB, VEC), jnp.float32),
        pltpu.VMEM((PAIRS, VEC), jnp.float32),
        pltpu.VMEM((PAIRS,), jnp.float32),
        pltpu.SemaphoreType.DMA,
        pltpu.SemaphoreType.DMA,
    ],
)
def _dbow_sc(doc_hbm, tn_hbm, d_tab_hbm, ot_tab_hbm, out_hbm,
             doc_v, tn_v, d_v, o_v, out_v, sem_d, sem_o):
    c = lax.axis_index("c")
    s = lax.axis_index("s")
    wid = s * 2 + c
    b0 = wid * BPW

    for sub in range(NSUB):
        pltpu.sync_copy(doc_hbm.at[pl.ds(b0 + sub * SUB, SUB)], doc_v)
        pltpu.sync_copy(
            tn_hbm.at[pl.ds(wid * ROWS_PER_W + sub * IDX_ROWS, IDX_ROWS)],
            tn_v)
        cps = [pltpu.async_copy(d_tab_hbm.at[doc_v], d_v, sem_d)]
        for r in range(IDX_ROWS):
            cps.append(
                pltpu.async_copy(ot_tab_hbm.at[tn_v.at[r]],
                                 o_v.at[pl.ds(r * 128, 128)], sem_o))
        for cp in cps:
            cp.wait()

        def ibody(i, carry):
            dv0 = d_v[i, pl.ds(0, 16)]
            dv1 = d_v[i, pl.ds(16, 16)]
            dv2 = d_v[i, pl.ds(32, 16)]
            dv3 = d_v[i, pl.ds(48, 16)]
            for k in range(K):
                j = i * K + k
                p = (dv0 * o_v[j, pl.ds(0, 16)]
                     + dv1 * o_v[j, pl.ds(16, 16)]
                     + dv2 * o_v[j, pl.ds(32, 16)]
                     + dv3 * o_v[j, pl.ds(48, 16)])
                out_v[j] = jnp.sum(p, axis=0)
            return carry

        lax.fori_loop(0, SUB, ibody, 0)
        pltpu.sync_copy(
            out_v, out_hbm.at[pl.ds((b0 + sub * SUB) * K, PAIRS)])


def kernel(context_ids, doc_ids, target_noise_ids, D, O):
    del context_ids
    ot = O.T
    doc = doc_ids.astype(jnp.int32)
    tn = target_noise_ids.astype(jnp.int32).reshape((B * K) // 128, 128)
    out_flat = _dbow_sc(doc, tn, D, ot)
    return out_flat.reshape(B, K)


# speedup vs baseline: 2.6126x; 2.6126x over previous
import functools

import jax
import jax.numpy as jnp
from jax import lax
from jax.experimental import pallas as pl
from jax.experimental.pallas import tpu as pltpu
from jax.experimental.pallas import tpu_sc as plsc

VEC = 64
B = 16384
K = 26
NW = 32
BPW = B // NW
SUB = 64
NSUB = BPW // SUB
PAIRS = SUB * K
IDX_ROWS = PAIRS // 128
IDX_ROWS_PAD = 16

_mesh = plsc.VectorSubcoreMesh(core_axis_name="c", subcore_axis_name="s")


@functools.partial(
    pl.kernel,
    mesh=_mesh,
    compiler_params=pltpu.CompilerParams(use_tc_tiling_on_sc=False),
    out_type=jax.ShapeDtypeStruct((B * K,), jnp.float32),
    scratch_types=[
        pltpu.VMEM((SUB,), jnp.int32),
        pltpu.VMEM((IDX_ROWS_PAD, 128), jnp.int32),
        pltpu.VMEM((SUB, VEC), jnp.float32),
        pltpu.VMEM((PAIRS, VEC), jnp.float32),
        pltpu.VMEM((PAIRS + 16,), jnp.float32),
        pltpu.SemaphoreType.DMA,
        pltpu.SemaphoreType.DMA,
    ],
)
def _dbow_sc(doc_hbm, tn_hbm, d_tab_hbm, ot_tab_hbm, out_hbm,
             doc_v, tn_v, d_v, o_v, out_v, sem_d, sem_o):
    c = lax.axis_index("c")
    s = lax.axis_index("s")
    wid = s * 2 + c
    b0 = wid * BPW

    for sub in range(NSUB):
        pltpu.sync_copy(doc_hbm.at[pl.ds(b0 + sub * SUB, SUB)], doc_v)
        pltpu.sync_copy(
            tn_hbm.at[pl.ds((wid * NSUB + sub) * IDX_ROWS_PAD, IDX_ROWS_PAD)],
            tn_v)
        cps = [pltpu.async_copy(d_tab_hbm.at[doc_v], d_v, sem_d)]
        for r in range(IDX_ROWS):
            cps.append(
                pltpu.async_copy(ot_tab_hbm.at[tn_v.at[r]],
                                 o_v.at[pl.ds(r * 128, 128)], sem_o))
        for cp in cps:
            cp.wait()

        lanes = lax.iota(jnp.int32, 16)
        perms = [lanes ^ (1 << t) for t in range(4)]
        dnums = lax.GatherDimensionNumbers(
            offset_dims=(), collapsed_slice_dims=(0,), start_index_map=(0,))

        def _shuf(x, perm):
            return lax.gather(x, perm[:, None], dnums, slice_sizes=(1,),
                              mode=lax.GatherScatterMode.PROMISE_IN_BOUNDS)

        def ibody(i, carry):
            dv0 = d_v[i, pl.ds(0, 16)]
            dv1 = d_v[i, pl.ds(16, 16)]
            dv2 = d_v[i, pl.ds(32, 16)]
            dv3 = d_v[i, pl.ds(48, 16)]
            acc0 = jnp.zeros((16,), jnp.float32)
            acc1 = jnp.zeros((16,), jnp.float32)
            j0 = i * K
            for k in range(K):
                j = j0 + k
                p = (dv0 * o_v[j, pl.ds(0, 16)]
                     + dv1 * o_v[j, pl.ds(16, 16)]
                     + dv2 * o_v[j, pl.ds(32, 16)]
                     + dv3 * o_v[j, pl.ds(48, 16)])
                for perm in perms:
                    p = p + _shuf(p, perm)
                if k < 16:
                    acc0 = jnp.where(lanes == k, p, acc0)
                else:
                    acc1 = jnp.where(lanes == (k - 16), p, acc1)
            out_v[pl.ds(j0, 16)] = acc0
            out_v[pl.ds(j0 + 16, 16)] = acc1
            return carry

        lax.fori_loop(0, SUB, ibody, 0)
        pltpu.sync_copy(
            out_v.at[pl.ds(0, PAIRS)],
            out_hbm.at[pl.ds((b0 + sub * SUB) * K, PAIRS)])


def kernel(context_ids, doc_ids, target_noise_ids, D, O):
    del context_ids
    ot = O.T
    doc = doc_ids.astype(jnp.int32)
    tn = target_noise_ids.astype(jnp.int32).reshape(NW * NSUB, IDX_ROWS, 128)
    tn = jnp.pad(tn, ((0, 0), (0, IDX_ROWS_PAD - IDX_ROWS), (0, 0)))
    tn = tn.reshape(NW * NSUB * IDX_ROWS_PAD, 128)
    out_flat = _dbow_sc(doc, tn, D, ot)
    return out_flat.reshape(B, K)


# speedup vs baseline: 2.6318x; 1.0074x over previous
import functools

import jax
import jax.numpy as jnp
from jax import lax
from jax.experimental import pallas as pl
from jax.experimental.pallas import tpu as pltpu
from jax.experimental.pallas import tpu_sc as plsc

VEC = 64
B = 16384
K = 26
NW = 32
BPW = B // NW
SUB = 64
NSUB = BPW // SUB
PAIRS = SUB * K
IDX_ROWS = PAIRS // 128
ROWS_PER_W = (B * K) // (NW * 128)
NUM_WORDS = 100000
TBLK = 512
TGRID = (NUM_WORDS + TBLK - 1) // TBLK

_mesh = plsc.VectorSubcoreMesh(core_axis_name="c", subcore_axis_name="s")


@functools.partial(
    pl.kernel,
    mesh=_mesh,
    compiler_params=pltpu.CompilerParams(use_tc_tiling_on_sc=False),
    out_type=jax.ShapeDtypeStruct((B * K,), jnp.float32),
    scratch_types=[
        pltpu.VMEM((SUB,), jnp.int32),
        pltpu.VMEM((IDX_ROWS, 128), jnp.int32),
        pltpu.VMEM((SUB, VEC), jnp.float32),
        pltpu.VMEM((PAIRS, VEC), jnp.float32),
        pltpu.VMEM((PAIRS + 16,), jnp.float32),
        pltpu.SemaphoreType.DMA,
        pltpu.SemaphoreType.DMA,
    ],
)
def _dbow_sc(doc_hbm, tn_hbm, d_tab_hbm, ot_tab_hbm, out_hbm,
             doc_v, tn_v, d_v, o_v, out_v, sem_d, sem_o):
    c = lax.axis_index("c")
    s = lax.axis_index("s")
    wid = s * 2 + c
    b0 = wid * BPW

    for sub in range(NSUB):
        pltpu.sync_copy(doc_hbm.at[pl.ds(b0 + sub * SUB, SUB)], doc_v)
        pltpu.sync_copy(
            tn_hbm.at[pl.ds(wid * ROWS_PER_W + sub * IDX_ROWS, IDX_ROWS)],
            tn_v)
        cps = [pltpu.async_copy(d_tab_hbm.at[doc_v], d_v, sem_d)]
        for r in range(IDX_ROWS):
            cps.append(
                pltpu.async_copy(ot_tab_hbm.at[tn_v.at[r]],
                                 o_v.at[pl.ds(r * 128, 128)], sem_o))
        for cp in cps:
            cp.wait()

        lanes = lax.iota(jnp.int32, 16)
        perms = [lanes ^ (1 << t) for t in range(4)]
        dnums = lax.GatherDimensionNumbers(
            offset_dims=(), collapsed_slice_dims=(0,), start_index_map=(0,))

        def _shuf(x, perm):
            return lax.gather(x, perm[:, None], dnums, slice_sizes=(1,),
                              mode=lax.GatherScatterMode.PROMISE_IN_BOUNDS)

        def ibody(i, carry):
            dv0 = d_v[i, pl.ds(0, 16)]
            dv1 = d_v[i, pl.ds(16, 16)]
            dv2 = d_v[i, pl.ds(32, 16)]
            dv3 = d_v[i, pl.ds(48, 16)]
            acc0 = jnp.zeros((16,), jnp.float32)
            acc1 = jnp.zeros((16,), jnp.float32)
            j0 = i * K
            for k in range(K):
                j = j0 + k
                p = (dv0 * o_v[j, pl.ds(0, 16)]
                     + dv1 * o_v[j, pl.ds(16, 16)]
                     + dv2 * o_v[j, pl.ds(32, 16)]
                     + dv3 * o_v[j, pl.ds(48, 16)])
                for perm in perms:
                    p = p + _shuf(p, perm)
                if k < 16:
                    acc0 = jnp.where(lanes == k, p, acc0)
                else:
                    acc1 = jnp.where(lanes == (k - 16), p, acc1)
            out_v[pl.ds(j0, 16)] = acc0
            out_v[pl.ds(j0 + 16, 16)] = acc1
            return carry

        lax.fori_loop(0, SUB, ibody, 0)
        pltpu.sync_copy(
            out_v.at[pl.ds(0, PAIRS)],
            out_hbm.at[pl.ds((b0 + sub * SUB) * K, PAIRS)])


def _transpose_body(o_ref, ot_ref):
    ot_ref[...] = o_ref[...].T


_transpose_tc = pl.pallas_call(
    _transpose_body,
    grid=(TGRID,),
    in_specs=[pl.BlockSpec((VEC, TBLK), lambda i: (0, i))],
    out_specs=pl.BlockSpec((TBLK, VEC), lambda i: (i, 0)),
    out_shape=jax.ShapeDtypeStruct((NUM_WORDS, VEC), jnp.float32),
)


def kernel(context_ids, doc_ids, target_noise_ids, D, O):
    del context_ids
    ot = _transpose_tc(O)
    doc = doc_ids.astype(jnp.int32)
    tn = target_noise_ids.astype(jnp.int32).reshape((B * K) // 128, 128)
    out_flat = _dbow_sc(doc, tn, D, ot)
    return out_flat.reshape(B, K)


# speedup vs baseline: 4.9898x; 1.8959x over previous
import functools

import jax
import jax.numpy as jnp
from jax import lax
from jax.experimental import pallas as pl
from jax.experimental.pallas import tpu as pltpu
from jax.experimental.pallas import tpu_sc as plsc

VEC = 64
B = 16384
K = 26
NUM_DOCS = 1000000
NUM_WORDS = 100000
NW = 32
BPW = B // NW
SUB = 64
NSUB = BPW // SUB
PAIRS = SUB * K
IDX_ROWS = PAIRS // 64
ROWS_PER_W = (B * K) // (NW * 64)
TBLK = 8192
DOC_SPLIT = (NUM_DOCS // 2 // TBLK) * TBLK
WORD_SPLIT = (NUM_WORDS // 2 // TBLK) * TBLK

_mesh = plsc.VectorSubcoreMesh(core_axis_name="c", subcore_axis_name="s")


def _halfpair_body(lo_ref, hi_ref, out_ref):
    eye = (lax.broadcasted_iota(jnp.int32, (VEC, VEC), 0)
           == lax.broadcasted_iota(jnp.int32, (VEC, VEC), 1)).astype(jnp.float32)
    dn = (((0,), (0,)), ((), ()))
    lo_t = lax.dot_general(lo_ref[...], eye, dn,
                           preferred_element_type=jnp.float32)
    hi_t = lax.dot_general(hi_ref[...], eye, dn,
                           preferred_element_type=jnp.float32)
    out_ref[...] = jnp.concatenate([lo_t, hi_t], axis=1)


def _make_halfpair(n_cols, split):
    nblk = -(-max(split, n_cols - split) // TBLK)
    hblk = split // TBLK
    return pl.pallas_call(
        _halfpair_body,
        grid=(nblk,),
        in_specs=[
            pl.BlockSpec((VEC, TBLK), lambda i: (0, i)),
            pl.BlockSpec((VEC, TBLK), lambda i, _h=hblk: (0, i + _h)),
        ],
        out_specs=pl.BlockSpec((TBLK, 2 * VEC), lambda i: (i, 0)),
        out_shape=jax.ShapeDtypeStruct((nblk * TBLK, 2 * VEC), jnp.float32),
    )


_halfpair_docs = _make_halfpair(NUM_DOCS, DOC_SPLIT)
_halfpair_words = _make_halfpair(NUM_WORDS, WORD_SPLIT)
DOC_TAB_ROWS = 2 * ((-(-max(DOC_SPLIT, NUM_DOCS - DOC_SPLIT) // TBLK)) * TBLK)
WORD_TAB_ROWS = 2 * ((-(-max(WORD_SPLIT, NUM_WORDS - WORD_SPLIT) // TBLK)) * TBLK)


@functools.partial(
    pl.kernel,
    mesh=_mesh,
    compiler_params=pltpu.CompilerParams(use_tc_tiling_on_sc=False),
    out_type=jax.ShapeDtypeStruct((B * K,), jnp.float32),
    scratch_types=[
        pltpu.VMEM((SUB,), jnp.int32),
        pltpu.VMEM((SUB,), jnp.int32),
        pltpu.VMEM((IDX_ROWS, 64), jnp.int32),
        pltpu.VMEM((IDX_ROWS, 64), jnp.int32),
        pltpu.VMEM((SUB, VEC), jnp.float32),
        pltpu.VMEM((PAIRS, VEC), jnp.float32),
        pltpu.VMEM((PAIRS + 16,), jnp.float32),
        pltpu.SemaphoreType.DMA,
        pltpu.SemaphoreType.DMA,
    ],
)
def _dbow_sc(doc_hbm, tn_hbm, d_tab_hbm, ot_tab_hbm, out_hbm,
             doc_v, docm_v, tn_v, tnm_v, d_v, o_v, out_v, sem_d, sem_o):
    c = lax.axis_index("c")
    s = lax.axis_index("s")
    wid = s * 2 + c
    b0 = wid * BPW

    lanes = lax.iota(jnp.int32, 16)
    perms = [lanes ^ (1 << t) for t in range(4)]
    dnums = lax.GatherDimensionNumbers(
        offset_dims=(), collapsed_slice_dims=(0,), start_index_map=(0,))

    def _shuf(x, perm):
        return lax.gather(x, perm[:, None], dnums, slice_sizes=(1,),
                          mode=lax.GatherScatterMode.PROMISE_IN_BOUNDS)

    for sub in range(NSUB):
        pltpu.sync_copy(doc_hbm.at[pl.ds(b0 + sub * SUB, SUB)], doc_v)
        pltpu.sync_copy(
            tn_hbm.at[pl.ds(wid * ROWS_PER_W + sub * IDX_ROWS, IDX_ROWS)],
            tn_v)
        for g in range(SUB // 16):
            v = doc_v[pl.ds(g * 16, 16)]
            hi = v >= DOC_SPLIT
            docm_v[pl.ds(g * 16, 16)] = (
                (v - jnp.where(hi, DOC_SPLIT, 0)) * 2 + jnp.where(hi, 1, 0))
        for r in range(IDX_ROWS):
            for g in range(4):
                v = tn_v[r, pl.ds(g * 16, 16)]
                hi = v >= WORD_SPLIT
                tnm_v[r, pl.ds(g * 16, 16)] = (
                    (v - jnp.where(hi, WORD_SPLIT, 0)) * 2
                    + jnp.where(hi, 1, 0))
        cps = [pltpu.async_copy(d_tab_hbm.at[docm_v], d_v, sem_d)]
        for r in range(IDX_ROWS):
            cps.append(
                pltpu.async_copy(ot_tab_hbm.at[tnm_v.at[r]],
                                 o_v.at[pl.ds(r * 64, 64)], sem_o))
        for cp in cps:
            cp.wait()

        def ibody(i, carry):
            dv0 = d_v[i, pl.ds(0, 16)]
            dv1 = d_v[i, pl.ds(16, 16)]
            dv2 = d_v[i, pl.ds(32, 16)]
            dv3 = d_v[i, pl.ds(48, 16)]
            acc0 = jnp.zeros((16,), jnp.float32)
            acc1 = jnp.zeros((16,), jnp.float32)
            j0 = i * K
            for k in range(K):
                j = j0 + k
                p = (dv0 * o_v[j, pl.ds(0, 16)]
                     + dv1 * o_v[j, pl.ds(16, 16)]
                     + dv2 * o_v[j, pl.ds(32, 16)]
                     + dv3 * o_v[j, pl.ds(48, 16)])
                for perm in perms:
                    p = p + _shuf(p, perm)
                if k < 16:
                    acc0 = jnp.where(lanes == k, p, acc0)
                else:
                    acc1 = jnp.where(lanes == (k - 16), p, acc1)
            out_v[pl.ds(j0, 16)] = acc0
            out_v[pl.ds(j0 + 16, 16)] = acc1
            return carry

        lax.fori_loop(0, SUB, ibody, 0)
        pltpu.sync_copy(
            out_v.at[pl.ds(0, PAIRS)],
            out_hbm.at[pl.ds((b0 + sub * SUB) * K, PAIRS)])


def kernel(context_ids, doc_ids, target_noise_ids, D, O):
    del context_ids
    dt = D.T
    dp = _halfpair_docs(dt, dt).reshape(DOC_TAB_ROWS, VEC)
    op = _halfpair_words(O, O).reshape(WORD_TAB_ROWS, VEC)
    doc = doc_ids.astype(jnp.int32)
    tn = target_noise_ids.astype(jnp.int32).reshape((B * K) // 64, 64)
    out_flat = _dbow_sc(doc, tn, dp, op)
    return out_flat.reshape(B, K)


# speedup vs baseline: 5.4025x; 1.0827x over previous
import functools

import jax
import jax.numpy as jnp
from jax import lax
from jax.experimental import pallas as pl
from jax.experimental.pallas import tpu as pltpu
from jax.experimental.pallas import tpu_sc as plsc

VEC = 64
B = 16384
K = 26
NUM_DOCS = 1000000
NUM_WORDS = 100000
NW = 32
BPW = B // NW
SUB = 32
NSUB = BPW // SUB
PAIRS = SUB * K
IDX_ROWS = PAIRS // 64
ROWS_PER_W = (B * K) // (NW * 64)
TBLK = 8192
DOC_SPLIT = (NUM_DOCS // 2 // TBLK) * TBLK
WORD_SPLIT = (NUM_WORDS // 2 // TBLK) * TBLK

_mesh = plsc.VectorSubcoreMesh(core_axis_name="c", subcore_axis_name="s")


def _halfpair_body(lo_ref, hi_ref, out_ref):
    eye = (lax.broadcasted_iota(jnp.int32, (VEC, VEC), 0)
           == lax.broadcasted_iota(jnp.int32, (VEC, VEC), 1)).astype(jnp.float32)
    dn = (((0,), (0,)), ((), ()))
    lo_t = lax.dot_general(lo_ref[...], eye, dn,
                           preferred_element_type=jnp.float32)
    hi_t = lax.dot_general(hi_ref[...], eye, dn,
                           preferred_element_type=jnp.float32)
    out_ref[...] = jnp.concatenate([lo_t, hi_t], axis=1)


def _make_halfpair(n_cols, split):
    nblk = -(-max(split, n_cols - split) // TBLK)
    hblk = split // TBLK
    return pl.pallas_call(
        _halfpair_body,
        grid=(nblk,),
        in_specs=[
            pl.BlockSpec((VEC, TBLK), lambda i: (0, i)),
            pl.BlockSpec((VEC, TBLK), lambda i, _h=hblk: (0, i + _h)),
        ],
        out_specs=pl.BlockSpec((TBLK, 2 * VEC), lambda i: (i, 0)),
        out_shape=jax.ShapeDtypeStruct((nblk * TBLK, 2 * VEC), jnp.float32),
    )


_halfpair_docs = _make_halfpair(NUM_DOCS, DOC_SPLIT)
_halfpair_words = _make_halfpair(NUM_WORDS, WORD_SPLIT)
DOC_TAB_ROWS = 2 * ((-(-max(DOC_SPLIT, NUM_DOCS - DOC_SPLIT) // TBLK)) * TBLK)
WORD_TAB_ROWS = 2 * ((-(-max(WORD_SPLIT, NUM_WORDS - WORD_SPLIT) // TBLK)) * TBLK)


@functools.partial(
    pl.kernel,
    mesh=_mesh,
    compiler_params=pltpu.CompilerParams(use_tc_tiling_on_sc=False),
    out_type=jax.ShapeDtypeStruct((B * K,), jnp.float32),
    scratch_types=[
        pltpu.VMEM((2, SUB), jnp.int32),
        pltpu.VMEM((2, SUB), jnp.int32),
        pltpu.VMEM((2, IDX_ROWS, 64), jnp.int32),
        pltpu.VMEM((2, IDX_ROWS, 64), jnp.int32),
        pltpu.VMEM((2, SUB, VEC), jnp.float32),
        pltpu.VMEM((2, PAIRS, VEC), jnp.float32),
        pltpu.VMEM((PAIRS + 16,), jnp.float32),
        pltpu.SemaphoreType.DMA,
        pltpu.SemaphoreType.DMA,
        pltpu.SemaphoreType.DMA,
        pltpu.SemaphoreType.DMA,
    ],
)
def _dbow_sc(doc_hbm, tn_hbm, d_tab_hbm, ot_tab_hbm, out_hbm,
             doc_v, docm_v, tn_v, tnm_v, d_v, o_v, out_v,
             sem_d0, sem_o0, sem_d1, sem_o1):
    c = lax.axis_index("c")
    s = lax.axis_index("s")
    wid = s * 2 + c
    b0 = wid * BPW
    sems = [(sem_d0, sem_o0), (sem_d1, sem_o1)]

    lanes = lax.iota(jnp.int32, 16)
    perms = [lanes ^ (1 << t) for t in range(4)]
    dnums = lax.GatherDimensionNumbers(
        offset_dims=(), collapsed_slice_dims=(0,), start_index_map=(0,))

    def _shuf(x, perm):
        return lax.gather(x, perm[:, None], dnums, slice_sizes=(1,),
                          mode=lax.GatherScatterMode.PROMISE_IN_BOUNDS)

    def _stage(subi, par):
        sem_d, sem_o = sems[par]
        pltpu.sync_copy(doc_hbm.at[pl.ds(b0 + subi * SUB, SUB)],
                        doc_v.at[par])
        pltpu.sync_copy(
            tn_hbm.at[pl.ds(wid * ROWS_PER_W + subi * IDX_ROWS, IDX_ROWS)],
            tn_v.at[par])
        for g in range(SUB // 16):
            v = doc_v[par, pl.ds(g * 16, 16)]
            hi = v >= DOC_SPLIT
            docm_v[par, pl.ds(g * 16, 16)] = (
                (v - jnp.where(hi, DOC_SPLIT, 0)) * 2 + jnp.where(hi, 1, 0))
        for r in range(IDX_ROWS):
            for g in range(4):
                v = tn_v[par, r, pl.ds(g * 16, 16)]
                hi = v >= WORD_SPLIT
                tnm_v[par, r, pl.ds(g * 16, 16)] = (
                    (v - jnp.where(hi, WORD_SPLIT, 0)) * 2
                    + jnp.where(hi, 1, 0))
        pltpu.async_copy(d_tab_hbm.at[docm_v.at[par]], d_v.at[par], sem_d)
        for r in range(IDX_ROWS):
            pltpu.async_copy(ot_tab_hbm.at[tnm_v.at[par, r]],
                             o_v.at[par, pl.ds(r * 64, 64)], sem_o)

    def _drain(par):
        sem_d, sem_o = sems[par]
        pltpu.make_async_copy(d_tab_hbm.at[docm_v.at[par]], d_v.at[par],
                              sem_d).wait()
        for r in range(IDX_ROWS):
            pltpu.make_async_copy(ot_tab_hbm.at[tnm_v.at[par, r]],
                                  o_v.at[par, pl.ds(r * 64, 64)],
                                  sem_o).wait()

    def _compute(subi, par):
        def ibody(i, carry):
            dv0 = d_v[par, i, pl.ds(0, 16)]
            dv1 = d_v[par, i, pl.ds(16, 16)]
            dv2 = d_v[par, i, pl.ds(32, 16)]
            dv3 = d_v[par, i, pl.ds(48, 16)]
            acc0 = jnp.zeros((16,), jnp.float32)
            acc1 = jnp.zeros((16,), jnp.float32)
            j0 = i * K
            for k in range(K):
                j = j0 + k
                p = (dv0 * o_v[par, j, pl.ds(0, 16)]
                     + dv1 * o_v[par, j, pl.ds(16, 16)]
                     + dv2 * o_v[par, j, pl.ds(32, 16)]
                     + dv3 * o_v[par, j, pl.ds(48, 16)])
                for perm in perms:
                    p = p + _shuf(p, perm)
                if k < 16:
                    acc0 = jnp.where(lanes == k, p, acc0)
                else:
                    acc1 = jnp.where(lanes == (k - 16), p, acc1)
            out_v[pl.ds(j0, 16)] = acc0
            out_v[pl.ds(j0 + 16, 16)] = acc1
            return carry

        lax.fori_loop(0, SUB, ibody, 0)
        pltpu.sync_copy(
            out_v.at[pl.ds(0, PAIRS)],
            out_hbm.at[pl.ds((b0 + subi * SUB) * K, PAIRS)])

    _stage(0, 0)

    def body(t, carry):
        even = 2 * t
        _stage(even + 1, 1)
        _drain(0)
        _compute(even, 0)
        _stage(jnp.minimum(even + 2, NSUB - 2), 0)
        _drain(1)
        _compute(even + 1, 1)
        return carry

    lax.fori_loop(0, NSUB // 2, body, 0)
    _drain(0)


def kernel(context_ids, doc_ids, target_noise_ids, D, O):
    del context_ids
    dt = D.T
    dp = _halfpair_docs(dt, dt).reshape(DOC_TAB_ROWS, VEC)
    op = _halfpair_words(O, O).reshape(WORD_TAB_ROWS, VEC)
    doc = doc_ids.astype(jnp.int32)
    tn = target_noise_ids.astype(jnp.int32).reshape((B * K) // 64, 64)
    out_flat = _dbow_sc(doc, tn, dp, op)
    return out_flat.reshape(B, K)


# speedup vs baseline: 5.4130x; 1.0019x over previous
import functools

import jax
import jax.numpy as jnp
from jax import lax
from jax.experimental import pallas as pl
from jax.experimental.pallas import tpu as pltpu
from jax.experimental.pallas import tpu_sc as plsc

VEC = 64
B = 16384
K = 26
NUM_DOCS = 1000000
NUM_WORDS = 100000
NW = 32
BPW = B // NW
SUB = 32
NSUB = BPW // SUB
PAIRS = SUB * K
IDX_ROWS = PAIRS // 64
ROWS_PER_W = (B * K) // (NW * 64)
TBLK_DOC = 16384
TBLK_WORD = 2048
DOC_SPLIT = (NUM_DOCS // 2 // TBLK_DOC) * TBLK_DOC
WORD_SPLIT = (NUM_WORDS // 2 // TBLK_WORD) * TBLK_WORD

_mesh = plsc.VectorSubcoreMesh(core_axis_name="c", subcore_axis_name="s")


def _halfpair_body(lo_ref, hi_ref, out_ref):
    eye = (lax.broadcasted_iota(jnp.int32, (VEC, VEC), 0)
           == lax.broadcasted_iota(jnp.int32, (VEC, VEC), 1)).astype(jnp.float32)
    dn = (((0,), (0,)), ((), ()))
    lo_t = lax.dot_general(lo_ref[...], eye, dn,
                           preferred_element_type=jnp.float32)
    hi_t = lax.dot_general(hi_ref[...], eye, dn,
                           preferred_element_type=jnp.float32)
    out_ref[...] = jnp.concatenate([lo_t, hi_t], axis=1)


def _make_halfpair(n_cols, split, tblk):
    nblk = -(-max(split, n_cols - split) // tblk)
    hblk = split // tblk
    return pl.pallas_call(
        _halfpair_body,
        grid=(nblk,),
        in_specs=[
            pl.BlockSpec((VEC, tblk), lambda i: (0, i)),
            pl.BlockSpec((VEC, tblk), lambda i, _h=hblk: (0, i + _h)),
        ],
        out_specs=pl.BlockSpec((tblk, 2 * VEC), lambda i: (i, 0)),
        out_shape=jax.ShapeDtypeStruct((nblk * tblk, 2 * VEC), jnp.float32),
    )


_halfpair_docs = _make_halfpair(NUM_DOCS, DOC_SPLIT, TBLK_DOC)
_halfpair_words = _make_halfpair(NUM_WORDS, WORD_SPLIT, TBLK_WORD)
DOC_TAB_ROWS = 2 * ((-(-max(DOC_SPLIT, NUM_DOCS - DOC_SPLIT) // TBLK_DOC))
                    * TBLK_DOC)
WORD_TAB_ROWS = 2 * ((-(-max(WORD_SPLIT, NUM_WORDS - WORD_SPLIT)
                        // TBLK_WORD)) * TBLK_WORD)


@functools.partial(
    pl.kernel,
    mesh=_mesh,
    compiler_params=pltpu.CompilerParams(use_tc_tiling_on_sc=False),
    out_type=jax.ShapeDtypeStruct((B * K,), jnp.float32),
    scratch_types=[
        pltpu.VMEM((2, SUB), jnp.int32),
        pltpu.VMEM((2, SUB), jnp.int32),
        pltpu.VMEM((2, IDX_ROWS, 64), jnp.int32),
        pltpu.VMEM((2, IDX_ROWS, 64), jnp.int32),
        pltpu.VMEM((2, SUB, VEC), jnp.float32),
        pltpu.VMEM((2, PAIRS, VEC), jnp.float32),
        pltpu.VMEM((PAIRS + 16,), jnp.float32),
        pltpu.SemaphoreType.DMA,
        pltpu.SemaphoreType.DMA,
        pltpu.SemaphoreType.DMA,
        pltpu.SemaphoreType.DMA,
    ],
)
def _dbow_sc(doc_hbm, tn_hbm, d_tab_hbm, ot_tab_hbm, out_hbm,
             doc_v, docm_v, tn_v, tnm_v, d_v, o_v, out_v,
             sem_d0, sem_o0, sem_d1, sem_o1):
    c = lax.axis_index("c")
    s = lax.axis_index("s")
    wid = s * 2 + c
    b0 = wid * BPW
    sems = [(sem_d0, sem_o0), (sem_d1, sem_o1)]

    lanes = lax.iota(jnp.int32, 16)
    perms = [lanes ^ (1 << t) for t in range(4)]
    dnums = lax.GatherDimensionNumbers(
        offset_dims=(), collapsed_slice_dims=(0,), start_index_map=(0,))

    def _shuf(x, perm):
        return lax.gather(x, perm[:, None], dnums, slice_sizes=(1,),
                          mode=lax.GatherScatterMode.PROMISE_IN_BOUNDS)

    def _stage(subi, par):
        sem_d, sem_o = sems[par]
        pltpu.sync_copy(doc_hbm.at[pl.ds(b0 + subi * SUB, SUB)],
                        doc_v.at[par])
        pltpu.sync_copy(
            tn_hbm.at[pl.ds(wid * ROWS_PER_W + subi * IDX_ROWS, IDX_ROWS)],
            tn_v.at[par])
        for g in range(SUB // 16):
            v = doc_v[par, pl.ds(g * 16, 16)]
            hi = v >= DOC_SPLIT
            docm_v[par, pl.ds(g * 16, 16)] = (
                (v - jnp.where(hi, DOC_SPLIT, 0)) * 2 + jnp.where(hi, 1, 0))
        for r in range(IDX_ROWS):
            for g in range(4):
                v = tn_v[par, r, pl.ds(g * 16, 16)]
                hi = v >= WORD_SPLIT
                tnm_v[par, r, pl.ds(g * 16, 16)] = (
                    (v - jnp.where(hi, WORD_SPLIT, 0)) * 2
                    + jnp.where(hi, 1, 0))
        pltpu.async_copy(d_tab_hbm.at[docm_v.at[par]], d_v.at[par], sem_d)
        for r in range(IDX_ROWS):
            pltpu.async_copy(ot_tab_hbm.at[tnm_v.at[par, r]],
                             o_v.at[par, pl.ds(r * 64, 64)], sem_o)

    def _drain(par):
        sem_d, sem_o = sems[par]
        pltpu.make_async_copy(d_tab_hbm.at[docm_v.at[par]], d_v.at[par],
                              sem_d).wait()
        for r in range(IDX_ROWS):
            pltpu.make_async_copy(ot_tab_hbm.at[tnm_v.at[par, r]],
                                  o_v.at[par, pl.ds(r * 64, 64)],
                                  sem_o).wait()

    def _compute(subi, par):
        def ibody(i, carry):
            dv0 = d_v[par, i, pl.ds(0, 16)]
            dv1 = d_v[par, i, pl.ds(16, 16)]
            dv2 = d_v[par, i, pl.ds(32, 16)]
            dv3 = d_v[par, i, pl.ds(48, 16)]
            acc0 = jnp.zeros((16,), jnp.float32)
            acc1 = jnp.zeros((16,), jnp.float32)
            j0 = i * K
            for k in range(K):
                j = j0 + k
                p = (dv0 * o_v[par, j, pl.ds(0, 16)]
                     + dv1 * o_v[par, j, pl.ds(16, 16)]
                     + dv2 * o_v[par, j, pl.ds(32, 16)]
                     + dv3 * o_v[par, j, pl.ds(48, 16)])
                for perm in perms:
                    p = p + _shuf(p, perm)
                if k < 16:
                    acc0 = jnp.where(lanes == k, p, acc0)
                else:
                    acc1 = jnp.where(lanes == (k - 16), p, acc1)
            out_v[pl.ds(j0, 16)] = acc0
            out_v[pl.ds(j0 + 16, 16)] = acc1
            return carry

        lax.fori_loop(0, SUB, ibody, 0)
        pltpu.sync_copy(
            out_v.at[pl.ds(0, PAIRS)],
            out_hbm.at[pl.ds((b0 + subi * SUB) * K, PAIRS)])

    _stage(0, 0)

    def body(t, carry):
        even = 2 * t
        _stage(even + 1, 1)
        _drain(0)
        _compute(even, 0)
        _stage(jnp.minimum(even + 2, NSUB - 2), 0)
        _drain(1)
        _compute(even + 1, 1)
        return carry

    lax.fori_loop(0, NSUB // 2, body, 0)
    _drain(0)


def kernel(context_ids, doc_ids, target_noise_ids, D, O):
    del context_ids
    dt = D.T
    dp = _halfpair_docs(dt, dt).reshape(DOC_TAB_ROWS, VEC)
    op = _halfpair_words(O, O).reshape(WORD_TAB_ROWS, VEC)
    doc = doc_ids.astype(jnp.int32)
    tn = target_noise_ids.astype(jnp.int32).reshape((B * K) // 64, 64)
    out_flat = _dbow_sc(doc, tn, dp, op)
    return out_flat.reshape(B, K)


# speedup vs baseline: 5.5104x; 1.0180x over previous
import functools

import jax
import jax.numpy as jnp
from jax import lax
from jax.experimental import pallas as pl
from jax.experimental.pallas import tpu as pltpu
from jax.experimental.pallas import tpu_sc as plsc

VEC = 64
B = 16384
K = 26
NUM_DOCS = 1000000
NUM_WORDS = 100000
NW = 32
BPW = B // NW
SUB = 32
NSUB = BPW // SUB
PAIRS = SUB * K
IDX_ROWS = PAIRS // 64
ROWS_PER_W = (B * K) // (NW * 64)
TBLK_DOC = 16384
TBLK_WORD = 8192
DOC_SPLIT = (NUM_DOCS // 2 // TBLK_DOC) * TBLK_DOC
WORD_SPLIT = (NUM_WORDS // 2 // TBLK_WORD) * TBLK_WORD

_mesh = plsc.VectorSubcoreMesh(core_axis_name="c", subcore_axis_name="s")


def _halfpair_body(lo_ref, hi_ref, out_ref):
    eye = (lax.broadcasted_iota(jnp.int32, (VEC, VEC), 0)
           == lax.broadcasted_iota(jnp.int32, (VEC, VEC), 1)).astype(jnp.float32)
    dn = (((0,), (0,)), ((), ()))
    lo_t = lax.dot_general(lo_ref[...], eye, dn,
                           preferred_element_type=jnp.float32)
    hi_t = lax.dot_general(hi_ref[...], eye, dn,
                           preferred_element_type=jnp.float32)
    out_ref[...] = jnp.concatenate([lo_t, hi_t], axis=1)


def _make_halfpair(n_cols, split, tblk):
    nblk = -(-max(split, n_cols - split) // tblk)
    hblk = split // tblk
    return pl.pallas_call(
        _halfpair_body,
        grid=(nblk,),
        in_specs=[
            pl.BlockSpec((VEC, tblk), lambda i: (0, i)),
            pl.BlockSpec((VEC, tblk), lambda i, _h=hblk: (0, i + _h)),
        ],
        out_specs=pl.BlockSpec((tblk, 2 * VEC), lambda i: (i, 0)),
        out_shape=jax.ShapeDtypeStruct((nblk * tblk, 2 * VEC), jnp.float32),
    )


_halfpair_docs = _make_halfpair(NUM_DOCS, DOC_SPLIT, TBLK_DOC)
_halfpair_words = _make_halfpair(NUM_WORDS, WORD_SPLIT, TBLK_WORD)
DOC_TAB_ROWS = 2 * ((-(-max(DOC_SPLIT, NUM_DOCS - DOC_SPLIT) // TBLK_DOC))
                    * TBLK_DOC)
WORD_TAB_ROWS = 2 * ((-(-max(WORD_SPLIT, NUM_WORDS - WORD_SPLIT)
                        // TBLK_WORD)) * TBLK_WORD)


@functools.partial(
    pl.kernel,
    mesh=_mesh,
    compiler_params=pltpu.CompilerParams(use_tc_tiling_on_sc=False),
    out_type=jax.ShapeDtypeStruct((B * K,), jnp.float32),
    scratch_types=[
        pltpu.VMEM((2, SUB), jnp.int32),
        pltpu.VMEM((2, SUB), jnp.int32),
        pltpu.VMEM((2, IDX_ROWS, 64), jnp.int32),
        pltpu.VMEM((2, IDX_ROWS, 64), jnp.int32),
        pltpu.VMEM((2, SUB, VEC), jnp.float32),
        pltpu.VMEM((2, PAIRS, VEC), jnp.float32),
        pltpu.VMEM((PAIRS + 16,), jnp.float32),
        pltpu.SemaphoreType.DMA,
        pltpu.SemaphoreType.DMA,
        pltpu.SemaphoreType.DMA,
        pltpu.SemaphoreType.DMA,
    ],
)
def _dbow_sc(doc_hbm, tn_hbm, d_tab_hbm, ot_tab_hbm, out_hbm,
             doc_v, docm_v, tn_v, tnm_v, d_v, o_v, out_v,
             sem_d0, sem_o0, sem_d1, sem_o1):
    c = lax.axis_index("c")
    s = lax.axis_index("s")
    wid = s * 2 + c
    b0 = wid * BPW
    sems = [(sem_d0, sem_o0), (sem_d1, sem_o1)]

    lanes = lax.iota(jnp.int32, 16)
    perms = [lanes ^ (1 << t) for t in range(4)]
    dnums = lax.GatherDimensionNumbers(
        offset_dims=(), collapsed_slice_dims=(0,), start_index_map=(0,))

    def _shuf(x, perm):
        return lax.gather(x, perm[:, None], dnums, slice_sizes=(1,),
                          mode=lax.GatherScatterMode.PROMISE_IN_BOUNDS)

    def _stage(subi, par):
        sem_d, sem_o = sems[par]
        pltpu.sync_copy(doc_hbm.at[pl.ds(b0 + subi * SUB, SUB)],
                        doc_v.at[par])
        pltpu.sync_copy(
            tn_hbm.at[pl.ds(wid * ROWS_PER_W + subi * IDX_ROWS, IDX_ROWS)],
            tn_v.at[par])
        for g in range(SUB // 16):
            v = doc_v[par, pl.ds(g * 16, 16)]
            hi = v >= DOC_SPLIT
            docm_v[par, pl.ds(g * 16, 16)] = (
                (v - jnp.where(hi, DOC_SPLIT, 0)) * 2 + jnp.where(hi, 1, 0))
        for r in range(IDX_ROWS):
            for g in range(4):
                v = tn_v[par, r, pl.ds(g * 16, 16)]
                hi = v >= WORD_SPLIT
                tnm_v[par, r, pl.ds(g * 16, 16)] = (
                    (v - jnp.where(hi, WORD_SPLIT, 0)) * 2
                    + jnp.where(hi, 1, 0))
        pltpu.async_copy(d_tab_hbm.at[docm_v.at[par]], d_v.at[par], sem_d)
        for r in range(IDX_ROWS):
            pltpu.async_copy(ot_tab_hbm.at[tnm_v.at[par, r]],
                             o_v.at[par, pl.ds(r * 64, 64)], sem_o)

    def _drain(par):
        sem_d, sem_o = sems[par]
        pltpu.make_async_copy(d_tab_hbm.at[docm_v.at[par]], d_v.at[par],
                              sem_d).wait()
        for r in range(IDX_ROWS):
            pltpu.make_async_copy(ot_tab_hbm.at[tnm_v.at[par, r]],
                                  o_v.at[par, pl.ds(r * 64, 64)],
                                  sem_o).wait()

    def _compute(subi, par):
        def ibody(i, carry):
            dv0 = d_v[par, i, pl.ds(0, 16)]
            dv1 = d_v[par, i, pl.ds(16, 16)]
            dv2 = d_v[par, i, pl.ds(32, 16)]
            dv3 = d_v[par, i, pl.ds(48, 16)]
            acc0 = jnp.zeros((16,), jnp.float32)
            acc1 = jnp.zeros((16,), jnp.float32)
            j0 = i * K
            for k in range(K):
                j = j0 + k
                p = (dv0 * o_v[par, j, pl.ds(0, 16)]
                     + dv1 * o_v[par, j, pl.ds(16, 16)]
                     + dv2 * o_v[par, j, pl.ds(32, 16)]
                     + dv3 * o_v[par, j, pl.ds(48, 16)])
                for perm in perms:
                    p = p + _shuf(p, perm)
                if k < 16:
                    acc0 = jnp.where(lanes == k, p, acc0)
                else:
                    acc1 = jnp.where(lanes == (k - 16), p, acc1)
            out_v[pl.ds(j0, 16)] = acc0
            out_v[pl.ds(j0 + 16, 16)] = acc1
            return carry

        lax.fori_loop(0, SUB, ibody, 0)
        pltpu.sync_copy(
            out_v.at[pl.ds(0, PAIRS)],
            out_hbm.at[pl.ds((b0 + subi * SUB) * K, PAIRS)])

    _stage(0, 0)

    def body(t, carry):
        even = 2 * t
        _stage(even + 1, 1)
        _drain(0)
        _compute(even, 0)
        _stage(jnp.minimum(even + 2, NSUB - 2), 0)
        _drain(1)
        _compute(even + 1, 1)
        return carry

    lax.fori_loop(0, NSUB // 2, body, 0)
    _drain(0)


def kernel(context_ids, doc_ids, target_noise_ids, D, O):
    del context_ids
    dt = D.T
    dp = _halfpair_docs(dt, dt).reshape(DOC_TAB_ROWS, VEC)
    op = _halfpair_words(O, O).reshape(WORD_TAB_ROWS, VEC)
    doc = doc_ids.astype(jnp.int32)
    tn = target_noise_ids.astype(jnp.int32).reshape((B * K) // 64, 64)
    out_flat = _dbow_sc(doc, tn, dp, op)
    return out_flat.reshape(B, K)
